# baseline (speedup 1.0000x reference)
import functools

import jax
import jax.numpy as jnp
from jax import lax
from jax.experimental import pallas as pl
from jax.experimental.pallas import tpu as pltpu
from jax.experimental.pallas import tpu_sc as plsc

_N = 16384
_V = 100
_NC = 1
_NS = 16
_NW = _NC * _NS
_BPW = _N // _NW
_L = 16


def kernel(energies, indices):
    mesh = plsc.VectorSubcoreMesh(core_axis_name="c", subcore_axis_name="s",
                                  num_cores=1)

    @functools.partial(
        pl.kernel,
        mesh=mesh,
        out_type=jax.ShapeDtypeStruct((_N,), jnp.float32),
        scratch_types=[
            pltpu.VMEM((_V,), jnp.float32),
            pltpu.VMEM((_BPW,), jnp.int32),
            pltpu.VMEM((_BPW,), jnp.float32),
            pltpu.SemaphoreType.DMA,
            pltpu.SemaphoreType.DMA,
            pltpu.SemaphoreType.DMA,
            pltpu.SemaphoreType.DMA,
        ],
        compiler_params=pltpu.CompilerParams(needs_layout_passes=False),
    )
    def k(tab_hbm, idx_hbm, out_hbm, tab_v, idx_v, out_v,
          sem_t, sem_i0, sem_i1, sem_o):
        wid = lax.axis_index("s") * _NC + lax.axis_index("c")
        base = wid * _BPW
        half = _BPW // 2
        tab_cp = pltpu.async_copy(tab_hbm, tab_v, sem_t)
        i0_cp = pltpu.async_copy(idx_hbm.at[pl.ds(base, half)],
                                 idx_v.at[pl.ds(0, half)], sem_i0)
        i1_cp = pltpu.async_copy(idx_hbm.at[pl.ds(base + half, half)],
                                 idx_v.at[pl.ds(half, half)], sem_i1)
        tab_cp.wait()
        i0_cp.wait()

        def body(j, carry):
            iv = idx_v[pl.ds(j * _L, _L)]
            out_v[pl.ds(j * _L, _L)] = plsc.load_gather(tab_v, [iv])
            return carry

        lax.fori_loop(0, half // _L, body, 0, unroll=4)
        o0_cp = pltpu.async_copy(out_v.at[pl.ds(0, half)],
                                 out_hbm.at[pl.ds(base, half)], sem_o)
        i1_cp.wait()
        lax.fori_loop(half // _L, _BPW // _L, body, 0, unroll=4)
        o1_cp = pltpu.async_copy(out_v.at[pl.ds(half, half)],
                                 out_hbm.at[pl.ds(base + half, half)], sem_o)
        o0_cp.wait()
        o1_cp.wait()

    return k(energies, indices)
